# baseline (speedup 1.0000x reference)
import functools

import jax
import jax.numpy as jnp
from jax import lax
from jax.experimental import pallas as pl
from jax.experimental.pallas import tpu as pltpu
from jax.experimental.pallas import tpu_sc as plsc

_B, _S, _H = 4, 4096, 4096
_E = 64
_K = 8
_G = 64
_T = 1024
_HC = _H // 2

_NC = 2
_NS = 16
_NW = _NC * _NS
_NIDX = _B * _S * _K
_ROWS_PER_TILE = _NIDX // 128 // _NW


def _gate_kernel(x0_ref, x1_ref, wt_ref, b_ref, idx_ref, w_ref, bins_ref,
                 ssum_ref, ssum_acc):
    pid = pl.program_id(0)
    nsteps = pl.num_programs(0)

    @pl.when(pid == 0)
    def _init():
        ssum_acc[...] = jnp.zeros_like(ssum_acc)

    wt = wt_ref[...]
    logits = (jnp.dot(x0_ref[...], wt[:_HC], preferred_element_type=jnp.float32)
              + jnp.dot(x1_ref[...], wt[_HC:], preferred_element_type=jnp.float32)
              + b_ref[...])

    ex = jnp.exp(logits)
    scores = ex / jnp.sum(ex, axis=-1, keepdims=True)

    iota = jax.lax.broadcasted_iota(jnp.int32, scores.shape, 1)
    rev = (jnp.float32(_E) - iota.astype(jnp.float32))
    vals = scores
    top_vals = []
    top_ridx = []
    for _ in range(_K):
        m = jnp.max(vals, axis=-1, keepdims=True)
        r = jnp.max(jnp.where(vals == m, rev, 0.0), axis=-1, keepdims=True)
        top_vals.append(m)
        top_ridx.append(r)
        vals = jnp.where(rev == r, -1.0, vals)

    tv = jnp.concatenate(top_vals, axis=1)
    tr = jnp.concatenate(top_ridx, axis=1)
    denom = jnp.sum(tv, axis=-1, keepdims=True) + 1e-20
    ti = (jnp.float32(_E) - tr).astype(jnp.int32)
    idx_ref[...] = ti
    w_ref[...] = tv / denom

    blocks_per_batch = _S // _T
    b = pid // blocks_per_batch
    bins_ref[...] = ti + _E * b
    ssum_acc[...] += ((jax.lax.broadcasted_iota(jnp.int32, (_B, _E), 0) == b)
                      .astype(jnp.float32) * jnp.sum(scores, axis=0, keepdims=True))

    @pl.when(pid == nsteps - 1)
    def _finish():
        ssum_ref[...] = ssum_acc[...]


def _sc_hist(bins_hbm, ones_hbm, zeros_hbm, parts_hbm, idx_v, ones_v, shared):
    cid = lax.axis_index("c")
    sid = lax.axis_index("s")
    wid = cid * _NS + sid

    @pl.when(sid == 0)
    def _zero():
        pltpu.sync_copy(zeros_hbm, shared)

    plsc.subcore_barrier()
    pltpu.sync_copy(bins_hbm.at[pl.ds(wid * _ROWS_PER_TILE, _ROWS_PER_TILE), :],
                    idx_v)
    pltpu.sync_copy(ones_hbm, ones_v)
    for j in range(_ROWS_PER_TILE):
        pltpu.sync_copy(ones_v, shared.at[idx_v.at[j]], add=True)
    plsc.subcore_barrier()

    @pl.when(sid == 0)
    def _out():
        pltpu.sync_copy(shared, parts_hbm.at[cid])


_sc_hist_call = pl.kernel(
    _sc_hist,
    out_type=jax.ShapeDtypeStruct((_NC, _B * _E), jnp.float32),
    mesh=plsc.VectorSubcoreMesh(core_axis_name="c", subcore_axis_name="s",
                                num_cores=_NC, num_subcores=_NS),
    scratch_types=[
        pltpu.VMEM((_ROWS_PER_TILE, 128), jnp.int32),
        pltpu.VMEM((128,), jnp.float32),
        pltpu.VMEM_SHARED((_B * _E,), jnp.float32),
    ],
)


def _aux_kernel(parts_ref, ssum_ref, aux_ref):
    ce = parts_ref[0] + parts_ref[1]
    scale = _G / (_S * _K * _S * _B)
    aux_ref[...] = (jnp.sum(ce * ssum_ref[...]) * scale).reshape(1, 1)


@jax.jit
def _run(x, weight, bias):
    hidden = x.reshape(-1, _H)
    wt = weight.T
    b2 = bias.reshape(1, _E)
    n = hidden.shape[0]
    nb = n // _T
    topk_idx, topk_weight, bins, ssum = pl.pallas_call(
        _gate_kernel,
        grid=(nb,),
        in_specs=[
            pl.BlockSpec((_T, _HC), lambda i: (i, 0)),
            pl.BlockSpec((_T, _HC), lambda i: (i, 1)),
            pl.BlockSpec((_H, _E), lambda i: (0, 0)),
            pl.BlockSpec((1, _E), lambda i: (0, 0)),
        ],
        out_specs=[
            pl.BlockSpec((_T, _K), lambda i: (i, 0)),
            pl.BlockSpec((_T, _K), lambda i: (i, 0)),
            pl.BlockSpec((_T, _K), lambda i: (i, 0)),
            pl.BlockSpec((_B, _E), lambda i: (0, 0)),
        ],
        out_shape=[
            jax.ShapeDtypeStruct((n, _K), jnp.int32),
            jax.ShapeDtypeStruct((n, _K), jnp.float32),
            jax.ShapeDtypeStruct((n, _K), jnp.int32),
            jax.ShapeDtypeStruct((_B, _E), jnp.float32),
        ],
        scratch_shapes=[
            pltpu.VMEM((_B, _E), jnp.float32),
        ],
        compiler_params=pltpu.CompilerParams(
            dimension_semantics=("arbitrary",),
        ),
    )(hidden, hidden, wt, b2)

    ones = jnp.ones((128,), jnp.float32)
    zeros = jnp.zeros((_B * _E,), jnp.float32)
    parts = _sc_hist_call(bins.reshape(_NIDX // 128, 128), ones, zeros)

    aux = pl.pallas_call(
        _aux_kernel,
        out_shape=jax.ShapeDtypeStruct((1, 1), jnp.float32),
    )(parts.reshape(_NC, _B, _E), ssum)
    return topk_idx, topk_weight, aux[0, 0]


def kernel(x, weight, bias):
    return _run(x, weight, bias)


# speedup vs baseline: 1.0008x; 1.0008x over previous
import functools

import jax
import jax.numpy as jnp
from jax import lax
from jax.experimental import pallas as pl
from jax.experimental.pallas import tpu as pltpu
from jax.experimental.pallas import tpu_sc as plsc

_B, _S, _H = 4, 4096, 4096
_E = 64
_K = 8
_G = 64
_T = 1024
_HC = _H // 2

_NC = 2
_NS = 16
_NW = _NC * _NS
_NIDX = _B * _S * _K
_ROWS_PER_TILE = _NIDX // 128 // _NW


def _gate_kernel(x0_ref, x1_ref, wt_ref, b_ref, idx_ref, w_ref, bins_ref,
                 ssum_ref, ssum_acc):
    pid = pl.program_id(0)
    nsteps = pl.num_programs(0)

    @pl.when(pid == 0)
    def _init():
        ssum_acc[...] = jnp.zeros_like(ssum_acc)

    wt = wt_ref[...]
    logits = (jnp.dot(x0_ref[...], wt[:_HC], preferred_element_type=jnp.float32)
              + jnp.dot(x1_ref[...], wt[_HC:], preferred_element_type=jnp.float32)
              + b_ref[...])

    ex = jnp.exp(logits)
    scores = ex / jnp.sum(ex, axis=-1, keepdims=True)

    iota = jax.lax.broadcasted_iota(jnp.int32, scores.shape, 1)
    rev = (jnp.float32(_E) - iota.astype(jnp.float32))
    vals = scores
    top_vals = []
    top_ridx = []
    for _ in range(_K):
        m = jnp.max(vals, axis=-1, keepdims=True)
        r = jnp.max(jnp.where(vals == m, rev, 0.0), axis=-1, keepdims=True)
        top_vals.append(m)
        top_ridx.append(r)
        vals = jnp.where(rev == r, -1.0, vals)

    tv = jnp.concatenate(top_vals, axis=1)
    tr = jnp.concatenate(top_ridx, axis=1)
    denom = jnp.sum(tv, axis=-1, keepdims=True) + 1e-20
    ti = (jnp.float32(_E) - tr).astype(jnp.int32)
    idx_ref[...] = ti
    w_ref[...] = tv / denom

    blocks_per_batch = _S // _T
    b = pid // blocks_per_batch
    bins_ref[...] = ti + _E * b
    ssum_acc[...] += ((jax.lax.broadcasted_iota(jnp.int32, (_B, _E), 0) == b)
                      .astype(jnp.float32) * jnp.sum(scores, axis=0, keepdims=True))

    @pl.when(pid == nsteps - 1)
    def _finish():
        ssum_ref[...] = ssum_acc[...]


def _sc_hist(bins_hbm, ones_hbm, zeros_hbm, parts_hbm, idx_v, ones_v, shared,
             sem):
    cid = lax.axis_index("c")
    sid = lax.axis_index("s")
    wid = cid * _NS + sid

    @pl.when(sid == 0)
    def _zero():
        pltpu.sync_copy(zeros_hbm, shared)

    plsc.subcore_barrier()
    pltpu.sync_copy(bins_hbm.at[pl.ds(wid * _ROWS_PER_TILE, _ROWS_PER_TILE), :],
                    idx_v)
    pltpu.sync_copy(ones_hbm, ones_v)
    descs = [pltpu.async_copy(ones_v, shared.at[idx_v.at[j]], sem, add=True)
             for j in range(_ROWS_PER_TILE)]
    for d in descs:
        d.wait()
    plsc.subcore_barrier()

    @pl.when(sid == 0)
    def _out():
        pltpu.sync_copy(shared, parts_hbm.at[cid])


_sc_hist_call = pl.kernel(
    _sc_hist,
    out_type=jax.ShapeDtypeStruct((_NC, _B * _E), jnp.float32),
    mesh=plsc.VectorSubcoreMesh(core_axis_name="c", subcore_axis_name="s",
                                num_cores=_NC, num_subcores=_NS),
    scratch_types=[
        pltpu.VMEM((_ROWS_PER_TILE, 128), jnp.int32),
        pltpu.VMEM((128,), jnp.float32),
        pltpu.VMEM_SHARED((_B * _E,), jnp.float32),
        pltpu.SemaphoreType.DMA,
    ],
)


def _aux_kernel(parts_ref, ssum_ref, aux_ref):
    ce = parts_ref[0] + parts_ref[1]
    scale = _G / (_S * _K * _S * _B)
    aux_ref[...] = (jnp.sum(ce * ssum_ref[...]) * scale).reshape(1, 1)


@jax.jit
def _run(x, weight, bias):
    hidden = x.reshape(-1, _H)
    wt = weight.T
    b2 = bias.reshape(1, _E)
    n = hidden.shape[0]
    nb = n // _T
    topk_idx, topk_weight, bins, ssum = pl.pallas_call(
        _gate_kernel,
        grid=(nb,),
        in_specs=[
            pl.BlockSpec((_T, _HC), lambda i: (i, 0)),
            pl.BlockSpec((_T, _HC), lambda i: (i, 1)),
            pl.BlockSpec((_H, _E), lambda i: (0, 0)),
            pl.BlockSpec((1, _E), lambda i: (0, 0)),
        ],
        out_specs=[
            pl.BlockSpec((_T, _K), lambda i: (i, 0)),
            pl.BlockSpec((_T, _K), lambda i: (i, 0)),
            pl.BlockSpec((_T, _K), lambda i: (i, 0)),
            pl.BlockSpec((_B, _E), lambda i: (0, 0)),
        ],
        out_shape=[
            jax.ShapeDtypeStruct((n, _K), jnp.int32),
            jax.ShapeDtypeStruct((n, _K), jnp.float32),
            jax.ShapeDtypeStruct((n, _K), jnp.int32),
            jax.ShapeDtypeStruct((_B, _E), jnp.float32),
        ],
        scratch_shapes=[
            pltpu.VMEM((_B, _E), jnp.float32),
        ],
        compiler_params=pltpu.CompilerParams(
            dimension_semantics=("arbitrary",),
        ),
    )(hidden, hidden, wt, b2)

    ones = jnp.ones((128,), jnp.float32)
    zeros = jnp.zeros((_B * _E,), jnp.float32)
    parts = _sc_hist_call(bins.reshape(_NIDX // 128, 128), ones, zeros)

    aux = pl.pallas_call(
        _aux_kernel,
        out_shape=jax.ShapeDtypeStruct((1, 1), jnp.float32),
    )(parts.reshape(_NC, _B, _E), ssum)
    return topk_idx, topk_weight, aux[0, 0]


def kernel(x, weight, bias):
    return _run(x, weight, bias)


# speedup vs baseline: 1.2428x; 1.2419x over previous
import jax
import jax.numpy as jnp
from jax.experimental import pallas as pl
from jax.experimental.pallas import tpu as pltpu

_B, _S, _H = 4, 4096, 4096
_E = 64
_K = 8
_G = 64
_T = 1024
_HC = _H // 2


def _gate_kernel(x0_ref, x1_ref, wt_ref, b_ref, idx_ref, w_ref, aux_ref,
                 cnt_acc, ssum_acc):
    pid = pl.program_id(0)
    nsteps = pl.num_programs(0)

    @pl.when(pid == 0)
    def _init():
        cnt_acc[...] = jnp.zeros_like(cnt_acc)
        ssum_acc[...] = jnp.zeros_like(ssum_acc)

    wt = wt_ref[...]
    logits = (jnp.dot(x0_ref[...], wt[:_HC], preferred_element_type=jnp.float32)
              + jnp.dot(x1_ref[...], wt[_HC:], preferred_element_type=jnp.float32)
              + b_ref[...])

    ex = jnp.exp(logits)
    scores = ex / jnp.sum(ex, axis=-1, keepdims=True)

    iota = jax.lax.broadcasted_iota(jnp.int32, scores.shape, 1)
    rev = (jnp.float32(_E) - iota.astype(jnp.float32))
    vals = scores
    top_vals = []
    top_ridx = []
    for _ in range(_K):
        m = jnp.max(vals, axis=-1, keepdims=True)
        r = jnp.max(jnp.where(vals == m, rev, 0.0), axis=-1, keepdims=True)
        top_vals.append(m)
        top_ridx.append(r)
        vals = jnp.where(rev == r, -1.0, vals)

    tv = jnp.concatenate(top_vals, axis=1)
    tr = jnp.concatenate(top_ridx, axis=1)
    denom = jnp.sum(tv, axis=-1, keepdims=True) + 1e-20
    idx_ref[...] = (jnp.float32(_E) - tr).astype(jnp.int32)
    w_ref[...] = tv / denom

    blocks_per_batch = _S // _T
    b = pid // blocks_per_batch
    selected = (vals < -0.5).astype(jnp.float32)
    cnt = jnp.sum(selected, axis=0, keepdims=True)
    ssum = jnp.sum(scores, axis=0, keepdims=True)
    rows = jax.lax.broadcasted_iota(jnp.int32, (_B, _E), 0)
    hit = (rows == b).astype(jnp.float32)
    cnt_acc[...] += hit * cnt
    ssum_acc[...] += hit * ssum

    @pl.when(pid == nsteps - 1)
    def _finish():
        scale = _G / (_S * _K * _S * _B)
        aux_ref[...] = (jnp.sum(cnt_acc[...] * ssum_acc[...]) * scale).reshape(1, 1)


@jax.jit
def _run(x, weight, bias):
    hidden = x.reshape(-1, _H)
    wt = weight.T
    b2 = bias.reshape(1, _E)
    n = hidden.shape[0]
    nb = n // _T
    topk_idx, topk_weight, aux = pl.pallas_call(
        _gate_kernel,
        grid=(nb,),
        in_specs=[
            pl.BlockSpec((_T, _HC), lambda i: (i, 0)),
            pl.BlockSpec((_T, _HC), lambda i: (i, 1)),
            pl.BlockSpec((_H, _E), lambda i: (0, 0)),
            pl.BlockSpec((1, _E), lambda i: (0, 0)),
        ],
        out_specs=[
            pl.BlockSpec((_T, _K), lambda i: (i, 0)),
            pl.BlockSpec((_T, _K), lambda i: (i, 0)),
            pl.BlockSpec((1, 1), lambda i: (0, 0)),
        ],
        out_shape=[
            jax.ShapeDtypeStruct((n, _K), jnp.int32),
            jax.ShapeDtypeStruct((n, _K), jnp.float32),
            jax.ShapeDtypeStruct((1, 1), jnp.float32),
        ],
        scratch_shapes=[
            pltpu.VMEM((_B, _E), jnp.float32),
            pltpu.VMEM((_B, _E), jnp.float32),
        ],
        compiler_params=pltpu.CompilerParams(
            dimension_semantics=("arbitrary",),
        ),
    )(hidden, hidden, wt, b2)
    return topk_idx, topk_weight, aux[0, 0]


def kernel(x, weight, bias):
    return _run(x, weight, bias)
